# Initial kernel scaffold; baseline (speedup 1.0000x reference)
import jax
import jax.numpy as jnp
from jax.experimental import pallas as pl


def kernel(x, edge_index, edge_attr, params):
    raise NotImplementedError("write your pallas kernel here")


# speedup vs baseline: 1.2211x; 1.2211x over previous
import functools

import jax
import jax.numpy as jnp
from jax import lax
from jax.experimental import pallas as pl
from jax.experimental.pallas import tpu as pltpu
from jax.experimental.pallas import tpu_sc as plsc

H = 6
C = 96
D = H * C
NC = 2
NS = 16
NW = NC * NS
EB = 128
AW = 128
NPH = 5120
TR = 5248
GW = 128


def _mesh():
    return plsc.VectorSubcoreMesh(core_axis_name="c", subcore_axis_name="s",
                                  num_cores=NC, num_subcores=NS)


def _ln(y, w, b, eps=1e-5):
    mu = jnp.mean(y, axis=-1, keepdims=True)
    var = jnp.mean((y - mu) ** 2, axis=-1, keepdims=True)
    return (y - mu) * lax.rsqrt(var + eps) * w + b


def _tc_mm(x, w, b=None, lnw=None, lnb=None, relu=False, br=2000):
    r, k = x.shape
    n = w.shape[1]
    assert r % br == 0
    have_b, have_ln = b is not None, lnw is not None

    def body(*refs):
        i = 0
        x_ref = refs[i]; i += 1
        w_ref = refs[i]; i += 1
        b_ref = refs[i] if have_b else None
        i += have_b
        lnw_ref = refs[i] if have_ln else None
        i += have_ln
        lnb_ref = refs[i] if have_ln else None
        i += have_ln
        out_ref = refs[i]
        y = jnp.dot(x_ref[...], w_ref[...], preferred_element_type=jnp.float32)
        if have_b:
            y = y + b_ref[...]
        if have_ln:
            y = _ln(y, lnw_ref[...], lnb_ref[...])
        if relu:
            y = jnp.maximum(y, 0.0)
        out_ref[...] = y

    ins = [x, w]
    specs = [pl.BlockSpec((br, k), lambda i: (i, 0)),
             pl.BlockSpec((k, n), lambda i: (0, 0))]
    if have_b:
        ins.append(b)
        specs.append(pl.BlockSpec((n,), lambda i: (0,)))
    if have_ln:
        ins += [lnw, lnb]
        specs += [pl.BlockSpec((n,), lambda i: (0,))] * 2
    return pl.pallas_call(
        body,
        grid=(r // br,),
        in_specs=specs,
        out_specs=pl.BlockSpec((br, n), lambda i: (i, 0)),
        out_shape=jax.ShapeDtypeStruct((r, n), jnp.float32),
    )(*ins)


def _tc_headmm(x, w, b=None, br=2000, out_w=C):
    r = x.shape[0]
    assert r % br == 0
    have_b = b is not None

    def body(*refs):
        x_ref, w_ref = refs[0], refs[1]
        b_ref = refs[2] if have_b else None
        out_ref = refs[-1]
        y = jnp.dot(x_ref[...], w_ref[...], preferred_element_type=jnp.float32)
        if have_b:
            y = y + b_ref[...]
        pad = jnp.zeros((br, out_w - C), jnp.float32) if out_w > C else None
        for hh in range(H):
            yh = y[:, C * hh:C * (hh + 1)]
            out_ref[hh] = yh if pad is None else jnp.concatenate([yh, pad], axis=1)

    ins = [x, w]
    specs = [pl.BlockSpec((br, C), lambda i: (i, 0)),
             pl.BlockSpec((C, D), lambda i: (0, 0))]
    if have_b:
        ins.append(b)
        specs.append(pl.BlockSpec((D,), lambda i: (0,)))
    return pl.pallas_call(
        body,
        grid=(r // br,),
        in_specs=specs,
        out_specs=pl.BlockSpec((H, br, out_w), lambda i: (0, i, 0)),
        out_shape=jax.ShapeDtypeStruct((H, r, out_w), jnp.float32),
    )(*ins)


def _tc_loopattr(la_slab, n, br=2000):
    def body(a_ref, out_ref):
        seg = a_ref[:, 0:C]
        deg = a_ref[:, C:C + 1]
        out_ref[...] = seg / jnp.maximum(deg, 1.0)

    return pl.pallas_call(
        body,
        grid=(n // br,),
        in_specs=[pl.BlockSpec((br, AW), lambda i: (i, 0))],
        out_specs=pl.BlockSpec((br, C), lambda i: (i, 0)),
        out_shape=jax.ShapeDtypeStruct((n, C), jnp.float32),
    )(la_slab)


def _tc_combine(acc, xlh, xrh, eelh, att, bias, lnw, lnb, h_in, use_elu, br=1000):
    n = h_in.shape[0]

    def body(acc_ref, xl_ref, xr_ref, eel_ref, att_ref, bias_ref,
             lnw_ref, lnb_ref, h_ref, out_ref):
        tot = jnp.zeros((br, C), jnp.float32)
        for hh in range(H):
            xl = xl_ref[hh][:, 0:C]
            s = xl + xr_ref[hh][:, 0:C] + eel_ref[hh]
            s = jnp.maximum(s, 0.0) + 0.2 * jnp.minimum(s, 0.0)
            alpha = jnp.sum(s * att_ref[hh], axis=-1, keepdims=True)
            ex = jnp.exp(alpha)
            num = acc_ref[hh, :, 0:C] + ex * xl
            den = acc_ref[hh, :, C:C + 1] + ex + 1e-16
            tot = tot + num / den
        cc = tot / H + bias_ref[...]
        if use_elu:
            cc = jnp.where(cc > 0, cc, jnp.exp(jnp.minimum(cc, 0.0)) - 1.0)
        cc = _ln(cc, lnw_ref[...], lnb_ref[...])
        out_ref[...] = h_ref[...] + cc

    return pl.pallas_call(
        body,
        grid=(n // br,),
        in_specs=[pl.BlockSpec((H, br, AW), lambda i: (0, i, 0)),
                  pl.BlockSpec((H, br, GW), lambda i: (0, i, 0)),
                  pl.BlockSpec((H, br, GW), lambda i: (0, i, 0)),
                  pl.BlockSpec((H, br, C), lambda i: (0, i, 0)),
                  pl.BlockSpec((H, C), lambda i: (0, 0)),
                  pl.BlockSpec((C,), lambda i: (0,)),
                  pl.BlockSpec((C,), lambda i: (0,)),
                  pl.BlockSpec((C,), lambda i: (0,)),
                  pl.BlockSpec((br, C), lambda i: (i, 0))],
        out_specs=pl.BlockSpec((br, C), lambda i: (i, 0)),
        out_shape=jax.ShapeDtypeStruct((n, C), jnp.float32),
    )(acc, xlh, xrh, eelh, att, bias, lnw, lnb, h_in)


def _tc_pred_mlp(g, lnp1w, lnp1b, wp2, bp2, lnp2w, lnp2b, wp3, bp3, br=2000):
    r = g.shape[0]

    def body(g_ref, l1w, l1b, w2, b2, l2w, l2b, w3, b3, out_ref):
        z = jnp.maximum(_ln(g_ref[...], l1w[...], l1b[...]), 0.0)
        z = jnp.dot(z, w2[...], preferred_element_type=jnp.float32) + b2[...]
        z = jnp.maximum(_ln(z, l2w[...], l2b[...]), 0.0)
        out_ref[...] = jnp.dot(z, w3[...], preferred_element_type=jnp.float32) + b3[...]

    k1 = g.shape[1]
    k2 = wp2.shape[1]
    return pl.pallas_call(
        body,
        grid=(r // br,),
        in_specs=[pl.BlockSpec((br, k1), lambda i: (i, 0)),
                  pl.BlockSpec((k1,), lambda i: (0,)),
                  pl.BlockSpec((k1,), lambda i: (0,)),
                  pl.BlockSpec((k1, k2), lambda i: (0, 0)),
                  pl.BlockSpec((k2,), lambda i: (0,)),
                  pl.BlockSpec((k2,), lambda i: (0,)),
                  pl.BlockSpec((k2,), lambda i: (0,)),
                  pl.BlockSpec((k2, 1), lambda i: (0, 0)),
                  pl.BlockSpec((1,), lambda i: (0,))],
        out_specs=pl.BlockSpec((br, 1), lambda i: (i, 0)),
        out_shape=jax.ShapeDtypeStruct((r, 1), jnp.float32),
    )(g, lnp1w, lnp1b, wp2, bp2, lnp2w, lnp2b, wp3, bp3)


def _zero_vmem(ref, rows, width):
    offs = list(range(0, width - 15, 16))
    if width % 16:
        offs.append(width - 16)

    def row(i, _):
        for off in offs:
            ref[i, pl.ds(off, 16)] = jnp.zeros((16,), jnp.float32)
        return 0

    lax.fori_loop(0, rows, row, 0)


def _worker_id():
    return lax.axis_index("s") * NC + lax.axis_index("c")


def _sc_gat(src, dst, xlh, xrh, eeh, att, e_raw, mode):
    e = src.shape[0]
    assert e % EB == 0
    nblk = e // EB
    zpw = TR // NS
    dpw = NPH // NS
    zchunks = [(0, EB), (EB, EB), (2 * EB, zpw - 2 * EB)]

    @functools.partial(
        pl.kernel,
        out_type=(jax.ShapeDtypeStruct((H, NC, NPH, AW), jnp.float32),
                  jax.ShapeDtypeStruct((NC, NPH, AW), jnp.float32)),
        mesh=_mesh(),
        compiler_params=pltpu.CompilerParams(needs_layout_passes=False),
        scratch_types=[
            pltpu.VMEM((EB,), jnp.int32),
            pltpu.VMEM((EB,), jnp.int32),
            pltpu.VMEM((EB,), jnp.int32),
            pltpu.VMEM((EB, GW), jnp.float32),
            pltpu.VMEM((EB, GW), jnp.float32),
            pltpu.VMEM((EB, C), jnp.float32),
            pltpu.VMEM((EB, AW), jnp.float32),
            pltpu.VMEM((1, C + 16), jnp.float32),
            pltpu.VMEM((EB, AW), jnp.float32),
            pltpu.VMEM((16,), jnp.int32),
            pltpu.VMEM_SHARED((TR, AW), jnp.float32),
            pltpu.SemaphoreType.DMA,
            pltpu.SemaphoreType.DMA,
            pltpu.SemaphoreType.DMA,
        ],
    )
    def k(src_hbm, dst_hbm, xlh_hbm, xrh_hbm, eeh_hbm, att_hbm, eraw_hbm,
          mode_hbm, out_hbm, la_hbm,
          srcs_v, dsts_v, dstsl_v, xlr_v, xrr_v, eer_v, y_v, att_v, zb_v,
          mode_v, acc_sh, sem1, sem2, sem3):
        cid = lax.axis_index("c")
        sid = lax.axis_index("s")
        nblk_w = (nblk - 1 - sid) // NS + 1
        lo = cid * NPH

        pltpu.sync_copy(mode_hbm, mode_v)
        _zero_vmem(zb_v, EB, AW)
        _zero_vmem(y_v, EB, AW)
        lanes = lax.iota(jnp.int32, 16)
        ones16 = jnp.ones((16,), jnp.float32)

        def zero_acc():
            for off, rows in zchunks:
                pltpu.sync_copy(zb_v.at[pl.ds(0, rows)],
                                acc_sh.at[pl.ds(sid * zpw + off, rows)])

        def clamp_dst():
            def cgrp(gi, _):
                v = dsts_v[pl.ds(gi * 16, 16)] - lo
                ok = (v >= 0) & (v < NPH)
                dstsl_v[pl.ds(gi * 16, 16)] = jnp.where(ok, v, NPH)
                return 0

            lax.fori_loop(0, EB // 16, cgrp, 0)

        for hh in range(H):
            zero_acc()
            plsc.subcore_barrier()

            pltpu.sync_copy(att_hbm.at[hh, 0], att_v.at[0, pl.ds(0, C)])
            xl_h = xlh_hbm.at[hh]
            xr_h = xrh_hbm.at[hh]
            ee_h = eeh_hbm.at[hh]

            def blk_body(bi, _, xl_h=xl_h, xr_h=xr_h, ee_h=ee_h):
                base = (sid + bi * NS) * EB
                pltpu.sync_copy(src_hbm.at[pl.ds(base, EB)], srcs_v)
                pltpu.sync_copy(dst_hbm.at[pl.ds(base, EB)], dsts_v)
                c1 = pltpu.async_copy(xl_h.at[srcs_v], xlr_v, sem1)
                c2 = pltpu.async_copy(xr_h.at[dsts_v], xrr_v, sem2)
                c3 = pltpu.async_copy(ee_h.at[pl.ds(base, EB)], eer_v, sem3)
                clamp_dst()
                c1.wait()
                c2.wait()
                c3.wait()

                def grp(gi, _):
                    eidx = gi * 16 + lanes

                    def c_in(ci, alpha):
                        cv = jnp.full((16,), ci, jnp.int32)
                        a = plsc.load_gather(xlr_v, [eidx, cv])
                        b = plsc.load_gather(xrr_v, [eidx, cv])
                        d = plsc.load_gather(eer_v, [eidx, cv])
                        s = a + b + d
                        s = jnp.maximum(s, 0.0) + 0.2 * jnp.minimum(s, 0.0)
                        ac = att_v[0, pl.ds(ci, 16)][0]
                        return alpha + ac * s

                    alpha = lax.fori_loop(0, C, c_in, jnp.zeros((16,), jnp.float32))
                    ex = jnp.exp(alpha)

                    def c_out(ci, _):
                        cv = jnp.full((16,), ci, jnp.int32)
                        a = plsc.load_gather(xlr_v, [eidx, cv])
                        plsc.store_scatter(y_v, [eidx, cv], ex * a)
                        return 0

                    lax.fori_loop(0, C, c_out, 0)
                    plsc.store_scatter(y_v, [eidx, jnp.full((16,), C, jnp.int32)], ex)
                    return 0

                lax.fori_loop(0, EB // 16, grp, 0)
                pltpu.sync_copy(y_v, acc_sh.at[dstsl_v], add=True)
                return 0

            lax.fori_loop(0, nblk_w, blk_body, 0)
            plsc.subcore_barrier()
            pltpu.sync_copy(acc_sh.at[pl.ds(sid * dpw, dpw)],
                            out_hbm.at[hh, cid, pl.ds(sid * dpw, dpw)])

        mode0 = mode_v[pl.ds(0, 16)][0]

        @pl.when(mode0 == 1)
        def _la_pass():
            plsc.subcore_barrier()
            zero_acc()
            plsc.subcore_barrier()

            def blk_body(bi, _):
                base = (sid + bi * NS) * EB
                pltpu.sync_copy(dst_hbm.at[pl.ds(base, EB)], dsts_v)
                c3 = pltpu.async_copy(eraw_hbm.at[pl.ds(base, EB)], eer_v, sem3)
                clamp_dst()
                c3.wait()

                def grp(gi, _):
                    eidx = gi * 16 + lanes

                    def c_out(ci, _):
                        cv = jnp.full((16,), ci, jnp.int32)
                        d = plsc.load_gather(eer_v, [eidx, cv])
                        plsc.store_scatter(y_v, [eidx, cv], d)
                        return 0

                    lax.fori_loop(0, C, c_out, 0)
                    plsc.store_scatter(y_v, [eidx, jnp.full((16,), C, jnp.int32)], ones16)
                    return 0

                lax.fori_loop(0, EB // 16, grp, 0)
                pltpu.sync_copy(y_v, acc_sh.at[dstsl_v], add=True)
                return 0

            lax.fori_loop(0, nblk_w, blk_body, 0)
            plsc.subcore_barrier()
            pltpu.sync_copy(acc_sh.at[pl.ds(sid * dpw, dpw)],
                            la_hbm.at[cid, pl.ds(sid * dpw, dpw)])

    return k(src, dst, xlh, xrh, eeh, att, e_raw, mode)


def _sc_pred_gather(src, dst, pre_u, pre_v, pre_w):
    e = src.shape[0]
    k_w = pre_u.shape[1]
    nblk = e // EB

    @functools.partial(
        pl.kernel,
        out_type=jax.ShapeDtypeStruct((e, k_w), jnp.float32),
        mesh=_mesh(),
        compiler_params=pltpu.CompilerParams(needs_layout_passes=False),
        scratch_types=[
            pltpu.VMEM((EB,), jnp.int32),
            pltpu.VMEM((EB,), jnp.int32),
            pltpu.VMEM((EB, k_w), jnp.float32),
            pltpu.VMEM((EB, k_w), jnp.float32),
            pltpu.VMEM((EB, k_w), jnp.float32),
            pltpu.VMEM((EB, k_w), jnp.float32),
            pltpu.SemaphoreType.DMA,
            pltpu.SemaphoreType.DMA,
            pltpu.SemaphoreType.DMA,
        ],
    )
    def k(src_hbm, dst_hbm, u_hbm, v_hbm, w_hbm, out_hbm,
          srcs_v, dsts_v, ur_v, vr_v, wr_v, g_v, sem1, sem2, sem3):
        cid = lax.axis_index("c")
        sid = lax.axis_index("s")
        wid = sid * NC + cid
        nblk_w = (nblk - 1 - wid) // NW + 1

        def blk_body(bi, _):
            base = (wid + bi * NW) * EB
            pltpu.sync_copy(src_hbm.at[pl.ds(base, EB)], srcs_v)
            pltpu.sync_copy(dst_hbm.at[pl.ds(base, EB)], dsts_v)
            c1 = pltpu.async_copy(u_hbm.at[srcs_v], ur_v, sem1)
            c2 = pltpu.async_copy(v_hbm.at[dsts_v], vr_v, sem2)
            c3 = pltpu.async_copy(w_hbm.at[pl.ds(base, EB)], wr_v, sem3)
            c1.wait()
            c2.wait()
            c3.wait()

            def row(i, _):
                for j in range(k_w // 16):
                    sl = pl.ds(j * 16, 16)
                    g_v[i, sl] = ur_v[i, sl] + vr_v[i, sl] + wr_v[i, sl]
                return 0

            lax.fori_loop(0, EB, row, 0)
            pltpu.sync_copy(g_v, out_hbm.at[pl.ds(base, EB)])
            return 0

        lax.fori_loop(0, nblk_w, blk_body, 0)

    return k(src, dst, pre_u, pre_v, pre_w)


def kernel(x, edge_index, edge_attr, params):
    p = params
    src = edge_index[0]
    dst = edge_index[1]

    h = _tc_mm(x, p['W_ne'], p['b_ne'], p['ln_ne_w'], p['ln_ne_b'],
               relu=True, br=1000)
    e = _tc_mm(edge_attr, p['W_ee'], p['b_ee'], p['ln_ee_w'], p['ln_ee_b'],
               relu=True, br=2000)

    la = None
    for i in (1, 2, 3):
        att = p['att%d' % i]
        xlh = _tc_headmm(h, p['Wl%d' % i], p['bl%d' % i], br=1000, out_w=GW)
        xrh = _tc_headmm(h, p['Wr%d' % i], p['br%d' % i], br=1000, out_w=GW)
        eeh = _tc_headmm(e, p['We%d' % i], br=2000)
        mode = jnp.full((16,), 1 if i == 1 else 0, jnp.int32)
        acc, la_slab = _sc_gat(src, dst, xlh, xrh, eeh,
                               att.reshape(H, 1, C), e, mode)
        acc = acc.reshape(H, NC * NPH, AW)
        if i == 1:
            la = _tc_loopattr(la_slab.reshape(NC * NPH, AW), x.shape[0])
        eelh = _tc_headmm(la, p['We%d' % i], br=1000)
        h = _tc_combine(acc, xlh, xrh, eelh, att, p['bias%d' % i],
                        p['lnw%d' % i], p['lnb%d' % i], h, use_elu=(i < 3))

    pre_u = _tc_mm(h, p['Wp1'][:C], br=1000)
    pre_v = _tc_mm(h, p['Wp1'][C:2 * C], br=1000)
    pre_w = _tc_mm(e, p['Wp1'][2 * C:], p['bp1'], br=2000)
    g = _sc_pred_gather(src, dst, pre_u, pre_v, pre_w)
    return _tc_pred_mlp(g, p['lnp1_w'], p['lnp1_b'], p['Wp2'], p['bp2'],
                        p['lnp2_w'], p['lnp2_b'], p['Wp3'], p['bp3'])


# speedup vs baseline: 1.3516x; 1.1069x over previous
import functools

import jax
import jax.numpy as jnp
from jax import lax
from jax.experimental import pallas as pl
from jax.experimental.pallas import tpu as pltpu
from jax.experimental.pallas import tpu_sc as plsc

H = 6
C = 96
D = H * C
NC = 2
NS = 16
NW = NC * NS
EB = 128
AW = 128
NPH = 5120
TR = 5248
GW = 128


def _mesh():
    return plsc.VectorSubcoreMesh(core_axis_name="c", subcore_axis_name="s",
                                  num_cores=NC, num_subcores=NS)


def _ln(y, w, b, eps=1e-5):
    mu = jnp.mean(y, axis=-1, keepdims=True)
    var = jnp.mean((y - mu) ** 2, axis=-1, keepdims=True)
    return (y - mu) * lax.rsqrt(var + eps) * w + b


def _tc_mm(x, w, b=None, lnw=None, lnb=None, relu=False, br=2000):
    r, k = x.shape
    n = w.shape[1]
    assert r % br == 0
    have_b, have_ln = b is not None, lnw is not None

    def body(*refs):
        i = 0
        x_ref = refs[i]; i += 1
        w_ref = refs[i]; i += 1
        b_ref = refs[i] if have_b else None
        i += have_b
        lnw_ref = refs[i] if have_ln else None
        i += have_ln
        lnb_ref = refs[i] if have_ln else None
        i += have_ln
        out_ref = refs[i]
        y = jnp.dot(x_ref[...], w_ref[...], preferred_element_type=jnp.float32)
        if have_b:
            y = y + b_ref[...]
        if have_ln:
            y = _ln(y, lnw_ref[...], lnb_ref[...])
        if relu:
            y = jnp.maximum(y, 0.0)
        out_ref[...] = y

    ins = [x, w]
    specs = [pl.BlockSpec((br, k), lambda i: (i, 0)),
             pl.BlockSpec((k, n), lambda i: (0, 0))]
    if have_b:
        ins.append(b)
        specs.append(pl.BlockSpec((n,), lambda i: (0,)))
    if have_ln:
        ins += [lnw, lnb]
        specs += [pl.BlockSpec((n,), lambda i: (0,))] * 2
    return pl.pallas_call(
        body,
        grid=(r // br,),
        in_specs=specs,
        out_specs=pl.BlockSpec((br, n), lambda i: (i, 0)),
        out_shape=jax.ShapeDtypeStruct((r, n), jnp.float32),
    )(*ins)


def _tc_headmm(x, w, b=None, br=2000, out_w=C):
    r = x.shape[0]
    assert r % br == 0
    have_b = b is not None

    def body(*refs):
        x_ref, w_ref = refs[0], refs[1]
        b_ref = refs[2] if have_b else None
        out_ref = refs[-1]
        y = jnp.dot(x_ref[...], w_ref[...], preferred_element_type=jnp.float32)
        if have_b:
            y = y + b_ref[...]
        pad = jnp.zeros((br, out_w - C), jnp.float32) if out_w > C else None
        for hh in range(H):
            yh = y[:, C * hh:C * (hh + 1)]
            out_ref[hh] = yh if pad is None else jnp.concatenate([yh, pad], axis=1)

    ins = [x, w]
    specs = [pl.BlockSpec((br, C), lambda i: (i, 0)),
             pl.BlockSpec((C, D), lambda i: (0, 0))]
    if have_b:
        ins.append(b)
        specs.append(pl.BlockSpec((D,), lambda i: (0,)))
    return pl.pallas_call(
        body,
        grid=(r // br,),
        in_specs=specs,
        out_specs=pl.BlockSpec((H, br, out_w), lambda i: (0, i, 0)),
        out_shape=jax.ShapeDtypeStruct((H, r, out_w), jnp.float32),
    )(*ins)


def _tc_loopattr(la_slab, n, br=2000):
    def body(a_ref, out_ref):
        seg = a_ref[:, 0:C]
        deg = a_ref[:, C:C + 1]
        out_ref[...] = seg / jnp.maximum(deg, 1.0)

    return pl.pallas_call(
        body,
        grid=(n // br,),
        in_specs=[pl.BlockSpec((br, AW), lambda i: (i, 0))],
        out_specs=pl.BlockSpec((br, C), lambda i: (i, 0)),
        out_shape=jax.ShapeDtypeStruct((n, C), jnp.float32),
    )(la_slab)


def _tc_combine(acc, xlh, xrh, eelh, att, bias, lnw, lnb, h_in, use_elu, br=1000):
    n = h_in.shape[0]

    def body(acc_ref, xl_ref, xr_ref, eel_ref, att_ref, bias_ref,
             lnw_ref, lnb_ref, h_ref, out_ref):
        tot = jnp.zeros((br, C), jnp.float32)
        for hh in range(H):
            xl = xl_ref[hh][:, 0:C]
            s = xl + xr_ref[hh][:, 0:C] + eel_ref[hh]
            s = jnp.maximum(s, 0.0) + 0.2 * jnp.minimum(s, 0.0)
            alpha = jnp.sum(s * att_ref[hh], axis=-1, keepdims=True)
            ex = jnp.exp(alpha)
            num = acc_ref[hh, :, 0:C] + ex * xl
            den = acc_ref[hh, :, C:C + 1] + ex + 1e-16
            tot = tot + num / den
        cc = tot / H + bias_ref[...]
        if use_elu:
            cc = jnp.where(cc > 0, cc, jnp.exp(jnp.minimum(cc, 0.0)) - 1.0)
        cc = _ln(cc, lnw_ref[...], lnb_ref[...])
        out_ref[...] = h_ref[...] + cc

    return pl.pallas_call(
        body,
        grid=(n // br,),
        in_specs=[pl.BlockSpec((H, br, AW), lambda i: (0, i, 0)),
                  pl.BlockSpec((H, br, GW), lambda i: (0, i, 0)),
                  pl.BlockSpec((H, br, GW), lambda i: (0, i, 0)),
                  pl.BlockSpec((H, br, C), lambda i: (0, i, 0)),
                  pl.BlockSpec((H, C), lambda i: (0, 0)),
                  pl.BlockSpec((C,), lambda i: (0,)),
                  pl.BlockSpec((C,), lambda i: (0,)),
                  pl.BlockSpec((C,), lambda i: (0,)),
                  pl.BlockSpec((br, C), lambda i: (i, 0))],
        out_specs=pl.BlockSpec((br, C), lambda i: (i, 0)),
        out_shape=jax.ShapeDtypeStruct((n, C), jnp.float32),
    )(acc, xlh, xrh, eelh, att, bias, lnw, lnb, h_in)


def _tc_pred_mlp(g, lnp1w, lnp1b, wp2, bp2, lnp2w, lnp2b, wp3, bp3, br=2000):
    r = g.shape[0]

    def body(g_ref, l1w, l1b, w2, b2, l2w, l2b, w3, b3, out_ref):
        z = jnp.maximum(_ln(g_ref[...], l1w[...], l1b[...]), 0.0)
        z = jnp.dot(z, w2[...], preferred_element_type=jnp.float32) + b2[...]
        z = jnp.maximum(_ln(z, l2w[...], l2b[...]), 0.0)
        out_ref[...] = jnp.dot(z, w3[...], preferred_element_type=jnp.float32) + b3[...]

    k1 = g.shape[1]
    k2 = wp2.shape[1]
    return pl.pallas_call(
        body,
        grid=(r // br,),
        in_specs=[pl.BlockSpec((br, k1), lambda i: (i, 0)),
                  pl.BlockSpec((k1,), lambda i: (0,)),
                  pl.BlockSpec((k1,), lambda i: (0,)),
                  pl.BlockSpec((k1, k2), lambda i: (0, 0)),
                  pl.BlockSpec((k2,), lambda i: (0,)),
                  pl.BlockSpec((k2,), lambda i: (0,)),
                  pl.BlockSpec((k2,), lambda i: (0,)),
                  pl.BlockSpec((k2, 1), lambda i: (0, 0)),
                  pl.BlockSpec((1,), lambda i: (0,))],
        out_specs=pl.BlockSpec((br, 1), lambda i: (i, 0)),
        out_shape=jax.ShapeDtypeStruct((r, 1), jnp.float32),
    )(g, lnp1w, lnp1b, wp2, bp2, lnp2w, lnp2b, wp3, bp3)


def _zero_vmem(ref, rows, width):
    offs = list(range(0, width - 15, 16))
    if width % 16:
        offs.append(width - 16)

    def row(i, _):
        for off in offs:
            ref[i, pl.ds(off, 16)] = jnp.zeros((16,), jnp.float32)
        return 0

    lax.fori_loop(0, rows, row, 0)


def _worker_id():
    return lax.axis_index("s") * NC + lax.axis_index("c")


def _sc_gat(src, dst, xlh, xrh, eeh, att, e_raw, mode):
    e = src.shape[0]
    assert e % EB == 0
    nblk = e // EB
    zpw = TR // NS
    dpw = NPH // NS
    zchunks = [(0, EB), (EB, EB), (2 * EB, zpw - 2 * EB)]

    @functools.partial(
        pl.kernel,
        out_type=(jax.ShapeDtypeStruct((H, NC, NPH, AW), jnp.float32),
                  jax.ShapeDtypeStruct((NC, NPH, AW), jnp.float32)),
        mesh=_mesh(),
        compiler_params=pltpu.CompilerParams(needs_layout_passes=False),
        scratch_types=[
            pltpu.VMEM((EB,), jnp.int32),
            pltpu.VMEM((EB,), jnp.int32),
            pltpu.VMEM((EB,), jnp.int32),
            pltpu.VMEM((EB, GW), jnp.float32),
            pltpu.VMEM((EB, GW), jnp.float32),
            pltpu.VMEM((EB, C), jnp.float32),
            pltpu.VMEM((EB, AW), jnp.float32),
            pltpu.VMEM((1, C + 16), jnp.float32),
            pltpu.VMEM((EB, AW), jnp.float32),
            pltpu.VMEM((16,), jnp.int32),
            pltpu.VMEM_SHARED((TR, AW), jnp.float32),
            pltpu.SemaphoreType.DMA,
            pltpu.SemaphoreType.DMA,
            pltpu.SemaphoreType.DMA,
        ],
    )
    def k(src_hbm, dst_hbm, xlh_hbm, xrh_hbm, eeh_hbm, att_hbm, eraw_hbm,
          mode_hbm, out_hbm, la_hbm,
          srcs_v, dsts_v, dstsl_v, xlr_v, xrr_v, eer_v, y_v, att_v, zb_v,
          mode_v, acc_sh, sem1, sem2, sem3):
        cid = lax.axis_index("c")
        sid = lax.axis_index("s")
        nblk_w = (nblk - 1 - sid) // NS + 1
        lo = cid * NPH

        pltpu.sync_copy(mode_hbm, mode_v)
        _zero_vmem(zb_v, EB, AW)
        _zero_vmem(y_v, EB, AW)
        lanes = lax.iota(jnp.int32, 16)
        ones16 = jnp.ones((16,), jnp.float32)

        def zero_acc():
            for off, rows in zchunks:
                pltpu.sync_copy(zb_v.at[pl.ds(0, rows)],
                                acc_sh.at[pl.ds(sid * zpw + off, rows)])

        def clamp_dst():
            def cgrp(gi, _):
                v = dsts_v[pl.ds(gi * 16, 16)] - lo
                ok = (v >= 0) & (v < NPH)
                dstsl_v[pl.ds(gi * 16, 16)] = jnp.where(ok, v, NPH)
                return 0

            lax.fori_loop(0, EB // 16, cgrp, 0)

        for hh in range(H):
            zero_acc()
            plsc.subcore_barrier()

            pltpu.sync_copy(att_hbm.at[hh, 0], att_v.at[0, pl.ds(0, C)])
            xl_h = xlh_hbm.at[hh]
            xr_h = xrh_hbm.at[hh]
            ee_h = eeh_hbm.at[hh]

            def blk_body(bi, _, xl_h=xl_h, xr_h=xr_h, ee_h=ee_h):
                base = (sid + bi * NS) * EB
                pltpu.sync_copy(src_hbm.at[pl.ds(base, EB)], srcs_v)
                pltpu.sync_copy(dst_hbm.at[pl.ds(base, EB)], dsts_v)
                c1 = pltpu.async_copy(xl_h.at[srcs_v], xlr_v, sem1)
                c2 = pltpu.async_copy(xr_h.at[dsts_v], xrr_v, sem2)
                c3 = pltpu.async_copy(ee_h.at[pl.ds(base, EB)], eer_v, sem3)
                clamp_dst()
                c1.wait()
                c2.wait()
                c3.wait()

                def grp(gi, _):
                    eidx = gi * 16 + lanes

                    def c_in(ci, carry):
                        a0, a1, a2, a3 = carry
                        cb = ci * 8
                        cv = jnp.full((16,), cb, jnp.int32)
                        acc = [a0, a1, a2, a3]
                        for j in range(8):
                            a = plsc.load_gather(xlr_v, [eidx, cv + j])
                            b = plsc.load_gather(xrr_v, [eidx, cv + j])
                            d = plsc.load_gather(eer_v, [eidx, cv + j])
                            sj = a + b + d
                            sj = jnp.maximum(sj, 0.0) + 0.2 * jnp.minimum(sj, 0.0)
                            ac = att_v[0, pl.ds(cb + j, 16)][0]
                            acc[j % 4] = acc[j % 4] + ac * sj
                        return tuple(acc)

                    z16 = jnp.zeros((16,), jnp.float32)
                    a0, a1, a2, a3 = lax.fori_loop(0, C // 8, c_in,
                                                   (z16, z16, z16, z16))
                    ex = jnp.exp((a0 + a1) + (a2 + a3))

                    def c_out(ci, _):
                        cv = jnp.full((16,), ci * 8, jnp.int32)
                        for j in range(8):
                            a = plsc.load_gather(xlr_v, [eidx, cv + j])
                            plsc.store_scatter(y_v, [eidx, cv + j], ex * a)
                        return 0

                    lax.fori_loop(0, C // 8, c_out, 0)
                    plsc.store_scatter(y_v, [eidx, jnp.full((16,), C, jnp.int32)], ex)
                    return 0

                lax.fori_loop(0, EB // 16, grp, 0)
                pltpu.sync_copy(y_v, acc_sh.at[dstsl_v], add=True)
                return 0

            lax.fori_loop(0, nblk_w, blk_body, 0)
            plsc.subcore_barrier()
            pltpu.sync_copy(acc_sh.at[pl.ds(sid * dpw, dpw)],
                            out_hbm.at[hh, cid, pl.ds(sid * dpw, dpw)])

        mode0 = mode_v[pl.ds(0, 16)][0]

        @pl.when(mode0 == 1)
        def _la_pass():
            plsc.subcore_barrier()
            zero_acc()
            plsc.subcore_barrier()

            def blk_body(bi, _):
                base = (sid + bi * NS) * EB
                pltpu.sync_copy(dst_hbm.at[pl.ds(base, EB)], dsts_v)
                c3 = pltpu.async_copy(eraw_hbm.at[pl.ds(base, EB)], eer_v, sem3)
                clamp_dst()
                c3.wait()

                def grp(gi, _):
                    eidx = gi * 16 + lanes

                    def c_out(ci, _):
                        cv = jnp.full((16,), ci * 8, jnp.int32)
                        for j in range(8):
                            d = plsc.load_gather(eer_v, [eidx, cv + j])
                            plsc.store_scatter(y_v, [eidx, cv + j], d)
                        return 0

                    lax.fori_loop(0, C // 8, c_out, 0)
                    plsc.store_scatter(y_v, [eidx, jnp.full((16,), C, jnp.int32)], ones16)
                    return 0

                lax.fori_loop(0, EB // 16, grp, 0)
                pltpu.sync_copy(y_v, acc_sh.at[dstsl_v], add=True)
                return 0

            lax.fori_loop(0, nblk_w, blk_body, 0)
            plsc.subcore_barrier()
            pltpu.sync_copy(acc_sh.at[pl.ds(sid * dpw, dpw)],
                            la_hbm.at[cid, pl.ds(sid * dpw, dpw)])

    return k(src, dst, xlh, xrh, eeh, att, e_raw, mode)


def _sc_pred_gather(src, dst, pre_u, pre_v, pre_w):
    e = src.shape[0]
    k_w = pre_u.shape[1]
    nblk = e // EB

    @functools.partial(
        pl.kernel,
        out_type=jax.ShapeDtypeStruct((e, k_w), jnp.float32),
        mesh=_mesh(),
        compiler_params=pltpu.CompilerParams(needs_layout_passes=False),
        scratch_types=[
            pltpu.VMEM((EB,), jnp.int32),
            pltpu.VMEM((EB,), jnp.int32),
            pltpu.VMEM((EB, k_w), jnp.float32),
            pltpu.VMEM((EB, k_w), jnp.float32),
            pltpu.VMEM((EB, k_w), jnp.float32),
            pltpu.VMEM((EB, k_w), jnp.float32),
            pltpu.SemaphoreType.DMA,
            pltpu.SemaphoreType.DMA,
            pltpu.SemaphoreType.DMA,
        ],
    )
    def k(src_hbm, dst_hbm, u_hbm, v_hbm, w_hbm, out_hbm,
          srcs_v, dsts_v, ur_v, vr_v, wr_v, g_v, sem1, sem2, sem3):
        cid = lax.axis_index("c")
        sid = lax.axis_index("s")
        wid = sid * NC + cid
        nblk_w = (nblk - 1 - wid) // NW + 1

        def blk_body(bi, _):
            base = (wid + bi * NW) * EB
            pltpu.sync_copy(src_hbm.at[pl.ds(base, EB)], srcs_v)
            pltpu.sync_copy(dst_hbm.at[pl.ds(base, EB)], dsts_v)
            c1 = pltpu.async_copy(u_hbm.at[srcs_v], ur_v, sem1)
            c2 = pltpu.async_copy(v_hbm.at[dsts_v], vr_v, sem2)
            c3 = pltpu.async_copy(w_hbm.at[pl.ds(base, EB)], wr_v, sem3)
            c1.wait()
            c2.wait()
            c3.wait()

            def row(i, _):
                for j in range(k_w // 16):
                    sl = pl.ds(j * 16, 16)
                    g_v[i, sl] = ur_v[i, sl] + vr_v[i, sl] + wr_v[i, sl]
                return 0

            lax.fori_loop(0, EB, row, 0)
            pltpu.sync_copy(g_v, out_hbm.at[pl.ds(base, EB)])
            return 0

        lax.fori_loop(0, nblk_w, blk_body, 0)

    return k(src, dst, pre_u, pre_v, pre_w)


def kernel(x, edge_index, edge_attr, params):
    p = params
    src = edge_index[0]
    dst = edge_index[1]

    h = _tc_mm(x, p['W_ne'], p['b_ne'], p['ln_ne_w'], p['ln_ne_b'],
               relu=True, br=1000)
    e = _tc_mm(edge_attr, p['W_ee'], p['b_ee'], p['ln_ee_w'], p['ln_ee_b'],
               relu=True, br=2000)

    la = None
    for i in (1, 2, 3):
        att = p['att%d' % i]
        xlh = _tc_headmm(h, p['Wl%d' % i], p['bl%d' % i], br=1000, out_w=GW)
        xrh = _tc_headmm(h, p['Wr%d' % i], p['br%d' % i], br=1000, out_w=GW)
        eeh = _tc_headmm(e, p['We%d' % i], br=2000)
        mode = jnp.full((16,), 1 if i == 1 else 0, jnp.int32)
        acc, la_slab = _sc_gat(src, dst, xlh, xrh, eeh,
                               att.reshape(H, 1, C), e, mode)
        acc = acc.reshape(H, NC * NPH, AW)
        if i == 1:
            la = _tc_loopattr(la_slab.reshape(NC * NPH, AW), x.shape[0])
        eelh = _tc_headmm(la, p['We%d' % i], br=1000)
        h = _tc_combine(acc, xlh, xrh, eelh, att, p['bias%d' % i],
                        p['lnw%d' % i], p['lnb%d' % i], h, use_elu=(i < 3))

    pre_u = _tc_mm(h, p['Wp1'][:C], br=1000)
    pre_v = _tc_mm(h, p['Wp1'][C:2 * C], br=1000)
    pre_w = _tc_mm(e, p['Wp1'][2 * C:], p['bp1'], br=2000)
    g = _sc_pred_gather(src, dst, pre_u, pre_v, pre_w)
    return _tc_pred_mlp(g, p['lnp1_w'], p['lnp1_b'], p['Wp2'], p['bp2'],
                        p['lnp2_w'], p['lnp2_b'], p['Wp3'], p['bp3'])
